# baseline (speedup 1.0000x reference)
import functools

import jax
import jax.numpy as jnp
from jax import lax
from jax.experimental import pallas as pl
from jax.experimental.pallas import tpu as pltpu
from jax.experimental.pallas import tpu_sc as plsc

NC = 2
NS = 16
L = 16
NW = NC * NS

CHUNK = 16384
NBUF = 2
TBL_PAD = 128


@functools.lru_cache(maxsize=None)
def _build_lookup(n: int):
    per_w = n // NW
    n_chunks = per_w // CHUNK
    assert per_w % CHUNK == 0

    mesh = plsc.VectorSubcoreMesh(core_axis_name="c", subcore_axis_name="s")

    @functools.partial(
        pl.kernel,
        mesh=mesh,
        out_type=jax.ShapeDtypeStruct((n,), jnp.uint32),
        compiler_params=pltpu.CompilerParams(needs_layout_passes=False),
        scratch_types=[
            [pltpu.VMEM((CHUNK,), jnp.uint32) for _ in range(NBUF)],
            [pltpu.VMEM((CHUNK,), jnp.uint32) for _ in range(NBUF)],
            pltpu.VMEM((TBL_PAD,), jnp.int32),
            pltpu.VMEM((L,), jnp.int32),
            [pltpu.SemaphoreType.DMA for _ in range(NBUF)],
            [pltpu.SemaphoreType.DMA for _ in range(NBUF)],
        ],
    )
    def lookup(an_hbm, tbl_hbm, minz_hbm, lo_hbm, in_bufs, out_bufs, tbl_v,
               minz_v, in_sems, out_sems):
        wid = lax.axis_index("s") * NC + lax.axis_index("c")
        base = wid * per_w

        pltpu.sync_copy(tbl_hbm, tbl_v)
        pltpu.sync_copy(minz_hbm, minz_v)
        minz = minz_v[...]

        def start_in(c):
            return pltpu.async_copy(
                an_hbm.at[pl.ds(base + c * CHUNK, CHUNK)],
                in_bufs[c % NBUF],
                in_sems[c % NBUF],
            )

        in_dmas = {0: start_in(0)}
        out_dmas = {}
        for c in range(n_chunks):
            b = c % NBUF
            if c + 1 < n_chunks:
                in_dmas[c + 1] = start_in(c + 1)
            in_dmas.pop(c).wait()
            if c >= NBUF:
                out_dmas.pop(c - NBUF).wait()
            in_b = in_bufs[b]
            out_b = out_bufs[b]

            @plsc.parallel_loop(
                jnp.int32(0), jnp.int32(CHUNK), jnp.int32(L), unroll=16
            )
            def vec_body(p):
                v = plsc.bitcast(in_b[pl.ds(p, L)], jnp.int32)
                t = plsc.load_gather(tbl_v, [v - minz])
                out_b[pl.ds(p, L)] = plsc.bitcast(t, jnp.uint32)

            out_dmas[c] = pltpu.async_copy(
                out_b,
                lo_hbm.at[pl.ds(base + c * CHUNK, CHUNK)],
                out_sems[b],
            )
        for c in sorted(out_dmas):
            out_dmas.pop(c).wait()

    return lookup


def kernel(atomic_numbers, Z_to_index, min_Z):
    n = atomic_numbers.shape[0]
    an_lo = atomic_numbers.astype(jnp.uint32)
    tbl32 = Z_to_index.astype(jnp.int32)
    tbl_pad = jnp.zeros((TBL_PAD,), jnp.int32).at[: tbl32.shape[0]].set(tbl32)
    minz_v = jnp.full((L,), min_Z.astype(jnp.int32), dtype=jnp.int32)
    lo = _build_lookup(n)(an_lo, tbl_pad, minz_v)
    return lo.astype(jnp.int64)


# speedup vs baseline: 1.0019x; 1.0019x over previous
import functools

import jax
import jax.numpy as jnp
from jax import lax
from jax.experimental import pallas as pl
from jax.experimental.pallas import tpu as pltpu
from jax.experimental.pallas import tpu_sc as plsc

NC = 2
NS = 16
L = 16
NW = NC * NS

CHUNK = 16384
NBUF = 2
TBL_PAD = 128


@functools.lru_cache(maxsize=None)
def _build_lookup(n: int):
    per_w = n // NW
    n_chunks = per_w // CHUNK
    assert per_w % CHUNK == 0

    mesh = plsc.VectorSubcoreMesh(core_axis_name="c", subcore_axis_name="s")

    @functools.partial(
        pl.kernel,
        mesh=mesh,
        out_type=jax.ShapeDtypeStruct((n,), jnp.uint32),
        compiler_params=pltpu.CompilerParams(needs_layout_passes=False),
        scratch_types=[
            [pltpu.VMEM((CHUNK,), jnp.uint32) for _ in range(NBUF)],
            [pltpu.VMEM((CHUNK,), jnp.uint32) for _ in range(NBUF)],
            pltpu.VMEM((TBL_PAD,), jnp.int32),
            pltpu.VMEM((L,), jnp.int32),
            [pltpu.SemaphoreType.DMA for _ in range(NBUF)],
            [pltpu.SemaphoreType.DMA for _ in range(NBUF)],
        ],
    )
    def lookup(an_hbm, tbl_hbm, minz_hbm, lo_hbm, in_bufs, out_bufs, tbl_v,
               minz_v, in_sems, out_sems):
        wid = lax.axis_index("s") * NC + lax.axis_index("c")
        base = wid * per_w

        pltpu.sync_copy(tbl_hbm, tbl_v)
        pltpu.sync_copy(minz_hbm, minz_v)
        minz = minz_v[...]

        def start_in(c):
            return pltpu.async_copy(
                an_hbm.at[pl.ds(base + c * CHUNK, CHUNK)],
                in_bufs[c % NBUF],
                in_sems[c % NBUF],
            )

        in_dmas = {0: start_in(0)}
        out_dmas = {}
        for c in range(n_chunks):
            b = c % NBUF
            if c + 1 < n_chunks:
                in_dmas[c + 1] = start_in(c + 1)
            in_dmas.pop(c).wait()
            if c >= NBUF:
                out_dmas.pop(c - NBUF).wait()
            in_b = in_bufs[b]
            out_b = out_bufs[b]

            @plsc.parallel_loop(
                jnp.int32(0), jnp.int32(CHUNK), jnp.int32(L), unroll=8
            )
            def vec_body(p):
                v = plsc.bitcast(in_b[pl.ds(p, L)], jnp.int32)
                t = plsc.load_gather(tbl_v, [v - minz])
                out_b[pl.ds(p, L)] = plsc.bitcast(t, jnp.uint32)

            out_dmas[c] = pltpu.async_copy(
                out_b,
                lo_hbm.at[pl.ds(base + c * CHUNK, CHUNK)],
                out_sems[b],
            )
        for c in sorted(out_dmas):
            out_dmas.pop(c).wait()

    return lookup


def kernel(atomic_numbers, Z_to_index, min_Z):
    n = atomic_numbers.shape[0]
    an_lo = atomic_numbers.astype(jnp.uint32)
    tbl32 = Z_to_index.astype(jnp.int32)
    tbl_pad = jnp.zeros((TBL_PAD,), jnp.int32).at[: tbl32.shape[0]].set(tbl32)
    minz_v = jnp.full((L,), min_Z.astype(jnp.int32), dtype=jnp.int32)
    lo = _build_lookup(n)(an_lo, tbl_pad, minz_v)
    return lo.astype(jnp.int64)
